# baseline (speedup 1.0000x reference)
import functools

import jax
import jax.numpy as jnp
from jax import lax
from jax.experimental import pallas as pl
from jax.experimental.pallas import tpu as pltpu
from jax.experimental.pallas import tpu_sc as plsc

EPS = 1e-10
LO = 0.6931
HI = 1.31340
NBINS = 256
WIDTH = (HI - LO) / NBINS
INV_WIDTH = NBINS / (HI - LO)

SUB = 8
LANE = 512
BLK = SUB * LANE


def _stable_sigmoid(x):
    e = jnp.exp(-jnp.abs(x))
    return jnp.where(x >= 0, 1.0 / (1.0 + e), e / (1.0 + e))


def _pass_a_body(x_ref, t_ref, ut_ref, key_ref, c_ref, hist_ref, cnt_ref,
                 carry_ref):
    b = pl.program_id(0)

    @pl.when(b == 0)
    def _init():
        hist_ref[...] = jnp.zeros_like(hist_ref)
        carry_ref[0] = 0.0
        carry_ref[1] = 0.0

    x = x_ref[0]
    t = t_ref[0].astype(jnp.float32)
    p = _stable_sigmoid(x)
    loss = p - p * t + jnp.log1p(jnp.exp(-p))
    negf = 1.0 - t
    key = jnp.where(negf > 0.5, loss, -1.0)
    key_ref[0] = key

    ut = ut_ref[...]
    incl = jnp.dot(negf, ut, preferred_element_type=jnp.float32)
    rowtot = incl[:, LANE - 1:LANE]
    r_i = lax.broadcasted_iota(jnp.int32, (SUB, SUB), 0)
    c_i = lax.broadcasted_iota(jnp.int32, (SUB, SUB), 1)
    slt = (c_i < r_i).astype(jnp.float32)
    rowpref = jnp.dot(slt, rowtot, preferred_element_type=jnp.float32)
    excl = incl - negf + rowpref
    c_ref[0] = (excl + carry_ref[0]).astype(jnp.int32)
    carry_ref[0] = carry_ref[0] + rowpref[SUB - 1, 0] + rowtot[SUB - 1, 0]
    carry_ref[1] = carry_ref[1] + jnp.sum(t)

    binid = jnp.clip(((key - LO) * INV_WIDTH).astype(jnp.int32), 0, NBINS - 1)
    hi_b = binid // 16
    lo_b = binid - hi_b * 16
    iota_h = lax.broadcasted_iota(jnp.int32, (16, LANE), 0)
    iota_l = lax.broadcasted_iota(jnp.int32, (LANE, 16), 1)
    acc = jnp.zeros((16, 16), jnp.float32)
    for r in range(SUB):
        hr = hi_b[r:r + 1, :]
        lr = lo_b[r:r + 1, :]
        nr = negf[r:r + 1, :]
        oh_hi = jnp.where(hr == iota_h, nr, 0.0)
        oh_lo = (lr.reshape(LANE, 1) == iota_l).astype(jnp.float32)
        acc = acc + jnp.dot(oh_hi, oh_lo, preferred_element_type=jnp.float32)
    hist_ref[...] = hist_ref[...] + acc

    @pl.when(b == pl.num_programs(0) - 1)
    def _fin():
        cnt_ref[...] = jnp.zeros((1, 1), jnp.float32) + carry_ref[1]


def _make_sc_scatter(S):
    info = plsc.get_sparse_core_info()
    nw = info.num_cores * info.num_subcores
    per_w = S // nw
    CHUNK = 2048
    nch = per_w // CHUNK
    assert per_w % CHUNK == 0
    mesh = plsc.VectorSubcoreMesh(core_axis_name="c", subcore_axis_name="s")

    @functools.partial(
        pl.kernel,
        mesh=mesh,
        out_type=jax.ShapeDtypeStruct((S,), jnp.float32),
        scratch_types=[
            pltpu.VMEM((CHUNK,), jnp.float32),
            pltpu.VMEM((CHUNK,), jnp.int32),
            pltpu.VMEM((16, 128), jnp.int32),
            pltpu.VMEM((16, 128), jnp.float32),
            pltpu.SemaphoreType.DMA,
        ],
    )
    def sc_scatter(key_hbm, c_hbm, out_hbm, kv, cv, idx2, val2, sem):
        cid = lax.axis_index("c")
        sid = lax.axis_index("s")
        wid = sid * info.num_cores + cid
        base = wid * per_w

        def chunk(ch, carry):
            off = base + ch * CHUNK
            pltpu.sync_copy(key_hbm.at[pl.ds(off, CHUNK)], kv)
            pltpu.sync_copy(c_hbm.at[pl.ds(off, CHUNK)], cv)
            for i in range(CHUNK // 16):
                k16 = kv[pl.ds(i * 16, 16)]
                c16 = cv[pl.ds(i * 16, 16)]
                mi = jnp.where(k16 >= 0.0, c16, S - 1)
                r = (i * 16) // 128
                col = (i * 16) % 128
                idx2[r, pl.ds(col, 16)] = mi
                val2[r, pl.ds(col, 16)] = k16
            cps = [pltpu.async_copy(val2.at[r], out_hbm.at[idx2.at[r]], sem)
                   for r in range(16)]
            for cp in cps:
                cp.wait()
            return carry

        lax.fori_loop(0, nch, chunk, 0, unroll=False)

    return sc_scatter


def _pass_d_body(x_ref, t_ref, kc_ref, thr_ref, lim_ref, ut_ref, acc_ref,
                 carry_ref):
    b = pl.program_id(0)

    @pl.when(b == 0)
    def _init():
        acc_ref[...] = jnp.zeros_like(acc_ref)
        carry_ref[0] = 0.0

    x = x_ref[0]
    t = t_ref[0].astype(jnp.float32)
    kc = kc_ref[0]
    p = _stable_sigmoid(x)
    loss = p - p * t + jnp.log1p(jnp.exp(-p))
    ps = 1.0 / (1.0 + jnp.exp(-p))
    posf = t

    n_neg = lim_ref[0]
    n_hns = lim_ref[1]

    r_i = lax.broadcasted_iota(jnp.int32, (SUB, LANE), 0).astype(jnp.float32)
    l_i = lax.broadcasted_iota(jnp.int32, (SUB, LANE), 1).astype(jnp.float32)
    fpos = b.astype(jnp.float32) * BLK + r_i * LANE + l_i

    negslot = fpos < n_neg
    valid = negslot & (kc > thr_ref[11])
    validf = jnp.where(valid, 1.0, 0.0)

    segn = jnp.zeros((SUB, LANE), jnp.float32)
    for k in range(11):
        segn = segn + jnp.where(kc <= thr_ref[k], 1.0, 0.0)

    ut = ut_ref[...]
    incl = jnp.dot(posf, ut, preferred_element_type=jnp.float32)
    rowtot = incl[:, LANE - 1:LANE]
    rr = lax.broadcasted_iota(jnp.int32, (SUB, SUB), 0)
    cc = lax.broadcasted_iota(jnp.int32, (SUB, SUB), 1)
    slt = (cc < rr).astype(jnp.float32)
    rowpref = jnp.dot(slt, rowtot, preferred_element_type=jnp.float32)
    pi = incl - posf + rowpref + carry_ref[0]
    carry_ref[0] = carry_ref[0] + rowpref[SUB - 1, 0] + rowtot[SUB - 1, 0]

    slotp = n_hns + pi
    segp = jnp.zeros((SUB, LANE), jnp.float32)
    for k in range(11):
        segp = segp + jnp.where(slotp >= lim_ref[2 + k], 1.0, 0.0)

    a_v = ps * t
    for s in range(12):
        wn = jnp.where(valid & (segn == s), 1.0, 0.0)
        wp = jnp.where((posf > 0.5) & (segp == s), 1.0, 0.0)
        w = wn + wp
        acc_ref[s:s + 1, :] += jnp.sum(a_v * w, axis=0, keepdims=True)
        acc_ref[12 + s:13 + s, :] += jnp.sum(ps * w, axis=0, keepdims=True)
        acc_ref[24 + s:25 + s, :] += jnp.sum(t * w, axis=0, keepdims=True)
    acc_ref[36:37, :] += jnp.sum(loss * (validf + posf), axis=0, keepdims=True)


def kernel(preds, targs):
    bsz, ch = preds.shape[0], preds.shape[1]
    bc = bsz * ch
    S = preds.size
    nblk = S // BLK
    assert S % BLK == 0

    x3 = preds.reshape(nblk, SUB, LANE)
    t3 = targs.reshape(nblk, SUB, LANE)

    li = lax.broadcasted_iota(jnp.int32, (LANE, LANE), 0)
    lj = lax.broadcasted_iota(jnp.int32, (LANE, LANE), 1)
    ut512 = (li <= lj).astype(jnp.float32)

    key3, c3, hist, npos_arr = pl.pallas_call(
        _pass_a_body,
        grid=(nblk,),
        in_specs=[
            pl.BlockSpec((1, SUB, LANE), lambda b: (b, 0, 0)),
            pl.BlockSpec((1, SUB, LANE), lambda b: (b, 0, 0)),
            pl.BlockSpec((LANE, LANE), lambda b: (0, 0)),
        ],
        out_specs=[
            pl.BlockSpec((1, SUB, LANE), lambda b: (b, 0, 0)),
            pl.BlockSpec((1, SUB, LANE), lambda b: (b, 0, 0)),
            pl.BlockSpec((16, 16), lambda b: (0, 0)),
            pl.BlockSpec((1, 1), lambda b: (0, 0)),
        ],
        out_shape=[
            jax.ShapeDtypeStruct((nblk, SUB, LANE), jnp.float32),
            jax.ShapeDtypeStruct((nblk, SUB, LANE), jnp.int32),
            jax.ShapeDtypeStruct((16, 16), jnp.float32),
            jax.ShapeDtypeStruct((1, 1), jnp.float32),
        ],
        scratch_shapes=[pltpu.SMEM((2,), jnp.float32)],
    )(x3, t3, ut512)

    n_pos = npos_arr[0, 0].astype(jnp.int32)
    n_neg = jnp.int32(S) - n_pos
    n_hns = jnp.where(
        n_pos == 0,
        jnp.floor(0.1 * n_neg.astype(jnp.float32)).astype(jnp.int32),
        jnp.minimum(n_pos * 3, n_neg),
    )
    n_total = n_hns + n_pos
    n_needed = jnp.mod(n_total, bc)
    n_final = n_total + n_needed
    m = jnp.maximum(n_final // bc, 1)

    hist_flat = hist.reshape(NBINS)
    suffix = jnp.cumsum(hist_flat[::-1])[::-1]
    suffix = jnp.concatenate([suffix, jnp.zeros(1, jnp.float32)])
    edges = LO + WIDTH * jnp.arange(NBINS + 1, dtype=jnp.float32)

    def thresh_for(bv):
        h = jnp.argmax(suffix <= bv.astype(jnp.float32))
        return edges[h]

    ks = []
    for k in range(1, 12):
        bv = k * m
        ks.append(jnp.where(bv >= n_hns, jnp.float32(-2.0), thresh_for(bv)))
    k_cut = jnp.where(n_hns >= n_neg, jnp.float32(-0.5), thresh_for(n_hns))
    thr = jnp.stack(ks + [k_cut] + [jnp.float32(0.0)] * 4)

    lim = jnp.concatenate([
        jnp.stack([n_neg.astype(jnp.float32), n_hns.astype(jnp.float32)]),
        (jnp.arange(1, 12, dtype=jnp.float32) * m.astype(jnp.float32)),
        jnp.zeros(3, jnp.float32),
    ])

    kc_flat = _make_sc_scatter(S)(key3.reshape(S), c3.reshape(S))
    kc3 = kc_flat.reshape(nblk, SUB, LANE)

    acc = pl.pallas_call(
        _pass_d_body,
        grid=(nblk,),
        in_specs=[
            pl.BlockSpec((1, SUB, LANE), lambda b: (b, 0, 0)),
            pl.BlockSpec((1, SUB, LANE), lambda b: (b, 0, 0)),
            pl.BlockSpec((1, SUB, LANE), lambda b: (b, 0, 0)),
            pl.BlockSpec(memory_space=pltpu.SMEM),
            pl.BlockSpec(memory_space=pltpu.SMEM),
            pl.BlockSpec((LANE, LANE), lambda b: (0, 0)),
        ],
        out_specs=pl.BlockSpec((48, LANE), lambda b: (0, 0)),
        out_shape=jax.ShapeDtypeStruct((48, LANE), jnp.float32),
        scratch_shapes=[pltpu.SMEM((1,), jnp.float32)],
    )(x3, t3, kc3, thr, lim, ut512)

    sums = jnp.sum(acc, axis=1)
    inter = sums[0:12]
    sump = sums[12:24]
    sumt = sums[24:36]
    lsum = sums[36]

    dice_loss = 1.0 - (2.0 * inter + EPS) / (sump + sumt + EPS)
    dice = jnp.mean(dice_loss)
    mean_l = lsum / n_final.astype(jnp.float32)
    return dice + mean_l


# speedup vs baseline: 116.6955x; 116.6955x over previous
import functools

import jax
import jax.numpy as jnp
from jax import lax
from jax.experimental import pallas as pl
from jax.experimental.pallas import tpu as pltpu
from jax.experimental.pallas import tpu_sc as plsc

EPS = 1e-10
LO = 0.6931
HI = 1.31340
NBINS = 256
WIDTH = (HI - LO) / NBINS
INV_WIDTH = NBINS / (HI - LO)

SUB = 8
LANE = 512
BLK = SUB * LANE


def _stable_sigmoid(x):
    e = jnp.exp(-jnp.abs(x))
    return jnp.where(x >= 0, 1.0 / (1.0 + e), e / (1.0 + e))


def _pass_a_body(x_ref, t_ref, ut_ref, key_ref, c_ref, cpre_ref, hist_ref,
                 cnt_ref, carry_ref):
    b = pl.program_id(0)

    @pl.when(b == 0)
    def _init():
        hist_ref[...] = jnp.zeros_like(hist_ref)
        carry_ref[0] = 0.0
        carry_ref[1] = 0.0

    x = x_ref[0]
    t = t_ref[0].astype(jnp.float32)
    p = _stable_sigmoid(x)
    loss = p - p * t + jnp.log1p(jnp.exp(-p))
    negf = 1.0 - t
    key = jnp.where(negf > 0.5, loss, -1.0)
    key_ref[0] = key

    ut = ut_ref[...]
    incl = jnp.dot(negf, ut, preferred_element_type=jnp.float32)
    rowtot = incl[:, LANE - 1:LANE]
    r_i = lax.broadcasted_iota(jnp.int32, (SUB, SUB), 0)
    c_i = lax.broadcasted_iota(jnp.int32, (SUB, SUB), 1)
    slt = (c_i < r_i).astype(jnp.float32)
    rowpref = jnp.dot(slt, rowtot, preferred_element_type=jnp.float32)
    excl = incl - negf + rowpref
    c_ref[0] = (excl + carry_ref[0]).astype(jnp.int32)
    cpre_ref[0] = jnp.zeros((1, 1), jnp.float32) + carry_ref[0]
    carry_ref[0] = carry_ref[0] + rowpref[SUB - 1, 0] + rowtot[SUB - 1, 0]
    carry_ref[1] = carry_ref[1] + jnp.sum(t)

    binid = jnp.clip(((key - LO) * INV_WIDTH).astype(jnp.int32), 0, NBINS - 1)
    hi_b = binid // 16
    lo_b = binid - hi_b * 16
    iota_h = lax.broadcasted_iota(jnp.int32, (16, LANE), 0)
    iota_l = lax.broadcasted_iota(jnp.int32, (LANE, 16), 1)
    acc = jnp.zeros((16, 16), jnp.float32)
    for r in range(SUB):
        hr = hi_b[r:r + 1, :]
        lr = lo_b[r:r + 1, :]
        nr = negf[r:r + 1, :]
        oh_hi = jnp.where(hr == iota_h, nr, 0.0)
        oh_lo = (lr.reshape(LANE, 1) == iota_l).astype(jnp.float32)
        acc = acc + jnp.dot(oh_hi, oh_lo, preferred_element_type=jnp.float32)
    hist_ref[...] = hist_ref[...] + acc

    @pl.when(b == pl.num_programs(0) - 1)
    def _fin():
        cnt_ref[...] = jnp.zeros((1, 1), jnp.float32) + carry_ref[1]


def _make_sc_compact(S):
    info = plsc.get_sparse_core_info()
    ns = info.num_subcores
    half = S // 2
    per_w = half // ns
    CHUNK = 2048
    nch = per_w // CHUNK
    assert per_w % CHUNK == 0
    SH = half + 16
    mesh = plsc.VectorSubcoreMesh(core_axis_name="c", subcore_axis_name="s")

    @functools.partial(
        pl.kernel,
        mesh=mesh,
        out_type=jax.ShapeDtypeStruct((2, SH), jnp.float32),
        scratch_types=[
            pltpu.VMEM((CHUNK,), jnp.float32),
            pltpu.VMEM((CHUNK,), jnp.int32),
            pltpu.VMEM((16, 128), jnp.int32),
            pltpu.VMEM((16, 128), jnp.float32),
            pltpu.VMEM((32,), jnp.int32),
            pltpu.VMEM_SHARED((SH,), jnp.float32),
            pltpu.SemaphoreType.DMA,
        ],
    )
    def sc_compact(key_hbm, c_hbm, bases_hbm, out_hbm, kv, cv, idx2, val2,
                   bv, shared, sem):
        cid = lax.axis_index("c")
        sid = lax.axis_index("s")
        base = cid * half + sid * per_w

        pltpu.sync_copy(bases_hbm, bv)
        vb = bv[pl.ds(cid * 16, 16)]

        def chunk(ch, carry):
            off = base + ch * CHUNK
            pltpu.sync_copy(key_hbm.at[pl.ds(off, CHUNK)], kv)
            pltpu.sync_copy(c_hbm.at[pl.ds(off, CHUNK)], cv)
            for i in range(CHUNK // 16):
                k16 = kv[pl.ds(i * 16, 16)]
                c16 = cv[pl.ds(i * 16, 16)]
                mi = jnp.where(k16 >= 0.0, c16 - vb, SH - 8)
                r = (i * 16) // 128
                col = (i * 16) % 128
                idx2[r, pl.ds(col, 16)] = mi
                val2[r, pl.ds(col, 16)] = k16
            cps = [pltpu.async_copy(val2.at[r], shared.at[idx2.at[r]], sem)
                   for r in range(16)]
            for cp in cps:
                cp.wait()
            return carry

        lax.fori_loop(0, nch, chunk, 0, unroll=False)
        plsc.subcore_barrier()

        @pl.when(sid == 0)
        def _writeback():
            pltpu.sync_copy(shared, out_hbm.at[cid])

    return sc_compact


def _pass_d_body(x_ref, t_ref, kc_ref, thr_ref, lim_ref, ut_ref, acc_ref,
                 carry_ref):
    b = pl.program_id(0)

    @pl.when(b == 0)
    def _init():
        acc_ref[...] = jnp.zeros_like(acc_ref)
        carry_ref[0] = 0.0

    x = x_ref[0]
    t = t_ref[0].astype(jnp.float32)
    kc = kc_ref[0]
    p = _stable_sigmoid(x)
    loss = p - p * t + jnp.log1p(jnp.exp(-p))
    ps = 1.0 / (1.0 + jnp.exp(-p))
    posf = t

    n_neg = lim_ref[0]
    n_hns = lim_ref[1]

    r_i = lax.broadcasted_iota(jnp.int32, (SUB, LANE), 0).astype(jnp.float32)
    l_i = lax.broadcasted_iota(jnp.int32, (SUB, LANE), 1).astype(jnp.float32)
    fpos = b.astype(jnp.float32) * BLK + r_i * LANE + l_i

    negslot = fpos < n_neg
    valid = negslot & (kc > thr_ref[11])
    validf = jnp.where(valid, 1.0, 0.0)

    segn = jnp.zeros((SUB, LANE), jnp.float32)
    for k in range(11):
        segn = segn + jnp.where(kc <= thr_ref[k], 1.0, 0.0)

    ut = ut_ref[...]
    incl = jnp.dot(posf, ut, preferred_element_type=jnp.float32)
    rowtot = incl[:, LANE - 1:LANE]
    rr = lax.broadcasted_iota(jnp.int32, (SUB, SUB), 0)
    cc = lax.broadcasted_iota(jnp.int32, (SUB, SUB), 1)
    slt = (cc < rr).astype(jnp.float32)
    rowpref = jnp.dot(slt, rowtot, preferred_element_type=jnp.float32)
    pi = incl - posf + rowpref + carry_ref[0]
    carry_ref[0] = carry_ref[0] + rowpref[SUB - 1, 0] + rowtot[SUB - 1, 0]

    slotp = n_hns + pi
    segp = jnp.zeros((SUB, LANE), jnp.float32)
    for k in range(11):
        segp = segp + jnp.where(slotp >= lim_ref[2 + k], 1.0, 0.0)

    a_v = ps * t
    for s in range(12):
        wn = jnp.where(valid & (segn == s), 1.0, 0.0)
        wp = jnp.where((posf > 0.5) & (segp == s), 1.0, 0.0)
        w = wn + wp
        acc_ref[s:s + 1, :] += jnp.sum(a_v * w, axis=0, keepdims=True)
        acc_ref[12 + s:13 + s, :] += jnp.sum(ps * w, axis=0, keepdims=True)
        acc_ref[24 + s:25 + s, :] += jnp.sum(t * w, axis=0, keepdims=True)
    acc_ref[36:37, :] += jnp.sum(loss * (validf + posf), axis=0, keepdims=True)


def kernel(preds, targs):
    bsz, ch = preds.shape[0], preds.shape[1]
    bc = bsz * ch
    S = preds.size
    nblk = S // BLK
    assert S % BLK == 0

    x3 = preds.reshape(nblk, SUB, LANE)
    t3 = targs.reshape(nblk, SUB, LANE)

    li = lax.broadcasted_iota(jnp.int32, (LANE, LANE), 0)
    lj = lax.broadcasted_iota(jnp.int32, (LANE, LANE), 1)
    ut512 = (li <= lj).astype(jnp.float32)

    key3, c3, cpre, hist, npos_arr = pl.pallas_call(
        _pass_a_body,
        grid=(nblk,),
        in_specs=[
            pl.BlockSpec((1, SUB, LANE), lambda b: (b, 0, 0)),
            pl.BlockSpec((1, SUB, LANE), lambda b: (b, 0, 0)),
            pl.BlockSpec((LANE, LANE), lambda b: (0, 0)),
        ],
        out_specs=[
            pl.BlockSpec((1, SUB, LANE), lambda b: (b, 0, 0)),
            pl.BlockSpec((1, SUB, LANE), lambda b: (b, 0, 0)),
            pl.BlockSpec((1, 1, 1), lambda b: (b, 0, 0)),
            pl.BlockSpec((16, 16), lambda b: (0, 0)),
            pl.BlockSpec((1, 1), lambda b: (0, 0)),
        ],
        out_shape=[
            jax.ShapeDtypeStruct((nblk, SUB, LANE), jnp.float32),
            jax.ShapeDtypeStruct((nblk, SUB, LANE), jnp.int32),
            jax.ShapeDtypeStruct((nblk, 1, 1), jnp.float32),
            jax.ShapeDtypeStruct((16, 16), jnp.float32),
            jax.ShapeDtypeStruct((1, 1), jnp.float32),
        ],
        scratch_shapes=[pltpu.SMEM((2,), jnp.float32)],
    )(x3, t3, ut512)

    n_pos = npos_arr[0, 0].astype(jnp.int32)
    n_neg = jnp.int32(S) - n_pos
    n_hns = jnp.where(
        n_pos == 0,
        jnp.floor(0.1 * n_neg.astype(jnp.float32)).astype(jnp.int32),
        jnp.minimum(n_pos * 3, n_neg),
    )
    n_total = n_hns + n_pos
    n_needed = jnp.mod(n_total, bc)
    n_final = n_total + n_needed
    m = jnp.maximum(n_final // bc, 1)

    hist_flat = hist.reshape(NBINS)
    suffix = jnp.cumsum(hist_flat[::-1])[::-1]
    suffix = jnp.concatenate([suffix, jnp.zeros(1, jnp.float32)])
    edges = LO + WIDTH * jnp.arange(NBINS + 1, dtype=jnp.float32)

    def thresh_for(bv):
        h = jnp.argmax(suffix <= bv.astype(jnp.float32))
        return edges[h]

    ks = []
    for k in range(1, 12):
        bv = k * m
        ks.append(jnp.where(bv >= n_hns, jnp.float32(-2.0), thresh_for(bv)))
    k_cut = jnp.where(n_hns >= n_neg, jnp.float32(-0.5), thresh_for(n_hns))
    thr = jnp.stack(ks + [k_cut] + [jnp.float32(0.0)] * 4)

    lim = jnp.concatenate([
        jnp.stack([n_neg.astype(jnp.float32), n_hns.astype(jnp.float32)]),
        (jnp.arange(1, 12, dtype=jnp.float32) * m.astype(jnp.float32)),
        jnp.zeros(3, jnp.float32),
    ])

    half = S // 2
    h_mid = cpre.reshape(nblk)[nblk // 2].astype(jnp.int32)
    bases = jnp.broadcast_to(jnp.stack([jnp.int32(0), h_mid])[:, None],
                             (2, 16)).astype(jnp.int32).reshape(32)
    kc2 = _make_sc_compact(S)(key3.reshape(S), c3.reshape(S), bases)
    SH = half + 16
    pad = S + 1024 - SH
    kc0 = jnp.concatenate([kc2[0], jnp.zeros(pad, jnp.float32)])
    kc1 = jnp.concatenate([kc2[1], jnp.zeros(pad, jnp.float32)])
    arS = jnp.arange(S + 1024, dtype=jnp.int32)
    kc_big = jnp.where(arS < h_mid, kc0, jnp.roll(kc1, h_mid))
    kc3 = kc_big[:S].reshape(nblk, SUB, LANE)

    acc = pl.pallas_call(
        _pass_d_body,
        grid=(nblk,),
        in_specs=[
            pl.BlockSpec((1, SUB, LANE), lambda b: (b, 0, 0)),
            pl.BlockSpec((1, SUB, LANE), lambda b: (b, 0, 0)),
            pl.BlockSpec((1, SUB, LANE), lambda b: (b, 0, 0)),
            pl.BlockSpec(memory_space=pltpu.SMEM),
            pl.BlockSpec(memory_space=pltpu.SMEM),
            pl.BlockSpec((LANE, LANE), lambda b: (0, 0)),
        ],
        out_specs=pl.BlockSpec((48, LANE), lambda b: (0, 0)),
        out_shape=jax.ShapeDtypeStruct((48, LANE), jnp.float32),
        scratch_shapes=[pltpu.SMEM((1,), jnp.float32)],
    )(x3, t3, kc3, thr, lim, ut512)

    sums = jnp.sum(acc, axis=1)
    inter = sums[0:12]
    sump = sums[12:24]
    sumt = sums[24:36]
    lsum = sums[36]

    dice_loss = 1.0 - (2.0 * inter + EPS) / (sump + sumt + EPS)
    dice = jnp.mean(dice_loss)
    mean_l = lsum / n_final.astype(jnp.float32)
    return dice + mean_l


# speedup vs baseline: 168.8339x; 1.4468x over previous
import functools

import jax
import jax.numpy as jnp
from jax import lax
from jax.experimental import pallas as pl
from jax.experimental.pallas import tpu as pltpu
from jax.experimental.pallas import tpu_sc as plsc

EPS = 1e-10
LO = 0.6931
HI = 1.31340
NBINS = 256
WIDTH = (HI - LO) / NBINS
INV_WIDTH = NBINS / (HI - LO)

SUB = 64
LANE = 512
BLK = SUB * LANE


def _stable_sigmoid(x):
    e = jnp.exp(-jnp.abs(x))
    return jnp.where(x >= 0, 1.0 / (1.0 + e), e / (1.0 + e))


def _pass_a_body(x_ref, t_ref, ut_ref, key_ref, c_ref, cpre_ref, hist_ref,
                 cnt_ref, carry_ref):
    b = pl.program_id(0)

    @pl.when(b == 0)
    def _init():
        hist_ref[...] = jnp.zeros_like(hist_ref)
        carry_ref[0] = 0.0
        carry_ref[1] = 0.0

    x = x_ref[0]
    t = t_ref[0].astype(jnp.float32)
    p = _stable_sigmoid(x)
    loss = p - p * t + jnp.log1p(jnp.exp(-p))
    negf = 1.0 - t
    key = jnp.where(negf > 0.5, loss, -1.0)
    key_ref[0] = key

    ut = ut_ref[...]
    incl = jnp.dot(negf, ut, preferred_element_type=jnp.float32)
    rowtot = incl[:, LANE - 1:LANE]
    r_i = lax.broadcasted_iota(jnp.int32, (SUB, SUB), 0)
    c_i = lax.broadcasted_iota(jnp.int32, (SUB, SUB), 1)
    slt = (c_i < r_i).astype(jnp.float32)
    rowpref = jnp.dot(slt, rowtot, preferred_element_type=jnp.float32)
    excl = incl - negf + rowpref
    c_ref[0] = (excl + carry_ref[0]).astype(jnp.int32)
    cpre_ref[0] = jnp.zeros((1, 1), jnp.float32) + carry_ref[0]
    carry_ref[0] = carry_ref[0] + rowpref[SUB - 1, 0] + rowtot[SUB - 1, 0]
    carry_ref[1] = carry_ref[1] + jnp.sum(t)

    binid = jnp.clip(((key - LO) * INV_WIDTH).astype(jnp.int32), 0, NBINS - 1)
    hi_b = binid // 16
    lo_b = binid - hi_b * 16
    iota_h = lax.broadcasted_iota(jnp.int32, (16, LANE), 0)
    iota_l = lax.broadcasted_iota(jnp.int32, (LANE, 16), 1)
    acc = jnp.zeros((16, 16), jnp.float32)
    for r in range(SUB):
        hr = hi_b[r:r + 1, :]
        lr = lo_b[r:r + 1, :]
        nr = negf[r:r + 1, :]
        oh_hi = jnp.where(hr == iota_h, nr, 0.0)
        oh_lo = (lr.reshape(LANE, 1) == iota_l).astype(jnp.float32)
        acc = acc + jnp.dot(oh_hi, oh_lo, preferred_element_type=jnp.float32)
    hist_ref[...] = hist_ref[...] + acc

    @pl.when(b == pl.num_programs(0) - 1)
    def _fin():
        cnt_ref[...] = jnp.zeros((1, 1), jnp.float32) + carry_ref[1]


def _make_sc_compact(S):
    info = plsc.get_sparse_core_info()
    ns = info.num_subcores
    half = S // 2
    per_w = half // ns
    CHUNK = 2048
    nch = per_w // CHUNK
    assert per_w % CHUNK == 0
    SH = half + 16
    mesh = plsc.VectorSubcoreMesh(core_axis_name="c", subcore_axis_name="s")

    @functools.partial(
        pl.kernel,
        mesh=mesh,
        out_type=jax.ShapeDtypeStruct((2, SH), jnp.float32),
        scratch_types=[
            pltpu.VMEM((CHUNK,), jnp.float32),
            pltpu.VMEM((CHUNK,), jnp.int32),
            pltpu.VMEM((16, 128), jnp.int32),
            pltpu.VMEM((16, 128), jnp.float32),
            pltpu.VMEM((32,), jnp.int32),
            pltpu.VMEM_SHARED((SH,), jnp.float32),
            pltpu.SemaphoreType.DMA,
        ],
    )
    def sc_compact(key_hbm, c_hbm, bases_hbm, out_hbm, kv, cv, idx2, val2,
                   bv, shared, sem):
        cid = lax.axis_index("c")
        sid = lax.axis_index("s")
        base = cid * half + sid * per_w

        pltpu.sync_copy(bases_hbm, bv)
        vb = bv[pl.ds(cid * 16, 16)]

        def chunk(ch, carry):
            off = base + ch * CHUNK
            pltpu.sync_copy(key_hbm.at[pl.ds(off, CHUNK)], kv)
            pltpu.sync_copy(c_hbm.at[pl.ds(off, CHUNK)], cv)
            for i in range(CHUNK // 16):
                k16 = kv[pl.ds(i * 16, 16)]
                c16 = cv[pl.ds(i * 16, 16)]
                mi = jnp.where(k16 >= 0.0, c16 - vb, SH - 8)
                r = (i * 16) // 128
                col = (i * 16) % 128
                idx2[r, pl.ds(col, 16)] = mi
                val2[r, pl.ds(col, 16)] = k16
            cps = [pltpu.async_copy(val2.at[r], shared.at[idx2.at[r]], sem)
                   for r in range(16)]
            for cp in cps:
                cp.wait()
            return carry

        lax.fori_loop(0, nch, chunk, 0, unroll=False)
        plsc.subcore_barrier()

        @pl.when(sid == 0)
        def _writeback():
            pltpu.sync_copy(shared, out_hbm.at[cid])

    return sc_compact


def _pass_d_body(x_ref, t_ref, kc_ref, thr_ref, lim_ref, ut_ref, acc_ref,
                 carry_ref):
    b = pl.program_id(0)

    @pl.when(b == 0)
    def _init():
        acc_ref[...] = jnp.zeros_like(acc_ref)
        carry_ref[0] = 0.0

    x = x_ref[0]
    t = t_ref[0].astype(jnp.float32)
    kc = kc_ref[0]
    p = _stable_sigmoid(x)
    loss = p - p * t + jnp.log1p(jnp.exp(-p))
    ps = 1.0 / (1.0 + jnp.exp(-p))
    posf = t

    n_neg = lim_ref[0]
    n_hns = lim_ref[1]

    r_i = lax.broadcasted_iota(jnp.int32, (SUB, LANE), 0).astype(jnp.float32)
    l_i = lax.broadcasted_iota(jnp.int32, (SUB, LANE), 1).astype(jnp.float32)
    fpos = b.astype(jnp.float32) * BLK + r_i * LANE + l_i

    negslot = fpos < n_neg
    valid = negslot & (kc > thr_ref[11])
    validf = jnp.where(valid, 1.0, 0.0)

    segn = jnp.zeros((SUB, LANE), jnp.float32)
    for k in range(11):
        segn = segn + jnp.where(kc <= thr_ref[k], 1.0, 0.0)

    ut = ut_ref[...]
    incl = jnp.dot(posf, ut, preferred_element_type=jnp.float32)
    rowtot = incl[:, LANE - 1:LANE]
    rr = lax.broadcasted_iota(jnp.int32, (SUB, SUB), 0)
    cc = lax.broadcasted_iota(jnp.int32, (SUB, SUB), 1)
    slt = (cc < rr).astype(jnp.float32)
    rowpref = jnp.dot(slt, rowtot, preferred_element_type=jnp.float32)
    pi = incl - posf + rowpref + carry_ref[0]
    carry_ref[0] = carry_ref[0] + rowpref[SUB - 1, 0] + rowtot[SUB - 1, 0]

    slotp = n_hns + pi
    segp = jnp.zeros((SUB, LANE), jnp.float32)
    for k in range(11):
        segp = segp + jnp.where(slotp >= lim_ref[2 + k], 1.0, 0.0)

    a_v = ps * t
    for s in range(12):
        wn = jnp.where(valid & (segn == s), 1.0, 0.0)
        wp = jnp.where((posf > 0.5) & (segp == s), 1.0, 0.0)
        w = wn + wp
        acc_ref[s:s + 1, :] += jnp.sum(a_v * w, axis=0, keepdims=True)
        acc_ref[12 + s:13 + s, :] += jnp.sum(ps * w, axis=0, keepdims=True)
        acc_ref[24 + s:25 + s, :] += jnp.sum(t * w, axis=0, keepdims=True)
    acc_ref[36:37, :] += jnp.sum(loss * (validf + posf), axis=0, keepdims=True)


def kernel(preds, targs):
    bsz, ch = preds.shape[0], preds.shape[1]
    bc = bsz * ch
    S = preds.size
    nblk = S // BLK
    assert S % BLK == 0

    x3 = preds.reshape(nblk, SUB, LANE)
    t3 = targs.reshape(nblk, SUB, LANE)

    li = lax.broadcasted_iota(jnp.int32, (LANE, LANE), 0)
    lj = lax.broadcasted_iota(jnp.int32, (LANE, LANE), 1)
    ut512 = (li <= lj).astype(jnp.float32)

    key3, c3, cpre, hist, npos_arr = pl.pallas_call(
        _pass_a_body,
        grid=(nblk,),
        in_specs=[
            pl.BlockSpec((1, SUB, LANE), lambda b: (b, 0, 0)),
            pl.BlockSpec((1, SUB, LANE), lambda b: (b, 0, 0)),
            pl.BlockSpec((LANE, LANE), lambda b: (0, 0)),
        ],
        out_specs=[
            pl.BlockSpec((1, SUB, LANE), lambda b: (b, 0, 0)),
            pl.BlockSpec((1, SUB, LANE), lambda b: (b, 0, 0)),
            pl.BlockSpec((1, 1, 1), lambda b: (b, 0, 0)),
            pl.BlockSpec((16, 16), lambda b: (0, 0)),
            pl.BlockSpec((1, 1), lambda b: (0, 0)),
        ],
        out_shape=[
            jax.ShapeDtypeStruct((nblk, SUB, LANE), jnp.float32),
            jax.ShapeDtypeStruct((nblk, SUB, LANE), jnp.int32),
            jax.ShapeDtypeStruct((nblk, 1, 1), jnp.float32),
            jax.ShapeDtypeStruct((16, 16), jnp.float32),
            jax.ShapeDtypeStruct((1, 1), jnp.float32),
        ],
        scratch_shapes=[pltpu.SMEM((2,), jnp.float32)],
    )(x3, t3, ut512)

    n_pos = npos_arr[0, 0].astype(jnp.int32)
    n_neg = jnp.int32(S) - n_pos
    n_hns = jnp.where(
        n_pos == 0,
        jnp.floor(0.1 * n_neg.astype(jnp.float32)).astype(jnp.int32),
        jnp.minimum(n_pos * 3, n_neg),
    )
    n_total = n_hns + n_pos
    n_needed = jnp.mod(n_total, bc)
    n_final = n_total + n_needed
    m = jnp.maximum(n_final // bc, 1)

    hist_flat = hist.reshape(NBINS)
    suffix = jnp.cumsum(hist_flat[::-1])[::-1]
    suffix = jnp.concatenate([suffix, jnp.zeros(1, jnp.float32)])
    edges = LO + WIDTH * jnp.arange(NBINS + 1, dtype=jnp.float32)

    def thresh_for(bv):
        h = jnp.argmax(suffix <= bv.astype(jnp.float32))
        return edges[h]

    ks = []
    for k in range(1, 12):
        bv = k * m
        ks.append(jnp.where(bv >= n_hns, jnp.float32(-2.0), thresh_for(bv)))
    k_cut = jnp.where(n_hns >= n_neg, jnp.float32(-0.5), thresh_for(n_hns))
    thr = jnp.stack(ks + [k_cut] + [jnp.float32(0.0)] * 4)

    lim = jnp.concatenate([
        jnp.stack([n_neg.astype(jnp.float32), n_hns.astype(jnp.float32)]),
        (jnp.arange(1, 12, dtype=jnp.float32) * m.astype(jnp.float32)),
        jnp.zeros(3, jnp.float32),
    ])

    half = S // 2
    h_mid = cpre.reshape(nblk)[nblk // 2].astype(jnp.int32)
    bases = jnp.broadcast_to(jnp.stack([jnp.int32(0), h_mid])[:, None],
                             (2, 16)).astype(jnp.int32).reshape(32)
    kc2 = _make_sc_compact(S)(key3.reshape(S), c3.reshape(S), bases)
    SH = half + 16
    pad = S + 1024 - SH
    kc0 = jnp.concatenate([kc2[0], jnp.zeros(pad, jnp.float32)])
    kc1 = jnp.concatenate([kc2[1], jnp.zeros(pad, jnp.float32)])
    arS = jnp.arange(S + 1024, dtype=jnp.int32)
    kc_big = jnp.where(arS < h_mid, kc0, jnp.roll(kc1, h_mid))
    kc3 = kc_big[:S].reshape(nblk, SUB, LANE)

    acc = pl.pallas_call(
        _pass_d_body,
        grid=(nblk,),
        in_specs=[
            pl.BlockSpec((1, SUB, LANE), lambda b: (b, 0, 0)),
            pl.BlockSpec((1, SUB, LANE), lambda b: (b, 0, 0)),
            pl.BlockSpec((1, SUB, LANE), lambda b: (b, 0, 0)),
            pl.BlockSpec(memory_space=pltpu.SMEM),
            pl.BlockSpec(memory_space=pltpu.SMEM),
            pl.BlockSpec((LANE, LANE), lambda b: (0, 0)),
        ],
        out_specs=pl.BlockSpec((48, LANE), lambda b: (0, 0)),
        out_shape=jax.ShapeDtypeStruct((48, LANE), jnp.float32),
        scratch_shapes=[pltpu.SMEM((1,), jnp.float32)],
    )(x3, t3, kc3, thr, lim, ut512)

    sums = jnp.sum(acc, axis=1)
    inter = sums[0:12]
    sump = sums[12:24]
    sumt = sums[24:36]
    lsum = sums[36]

    dice_loss = 1.0 - (2.0 * inter + EPS) / (sump + sumt + EPS)
    dice = jnp.mean(dice_loss)
    mean_l = lsum / n_final.astype(jnp.float32)
    return dice + mean_l


# speedup vs baseline: 172.6310x; 1.0225x over previous
import functools

import jax
import jax.numpy as jnp
from jax import lax
from jax.experimental import pallas as pl
from jax.experimental.pallas import tpu as pltpu
from jax.experimental.pallas import tpu_sc as plsc

EPS = 1e-10
LO = 0.6931
HI = 1.31340
NBINS = 256
WIDTH = (HI - LO) / NBINS
INV_WIDTH = NBINS / (HI - LO)

SUB = 128
LANE = 512
BLK = SUB * LANE


def _stable_sigmoid(x):
    e = jnp.exp(-jnp.abs(x))
    return jnp.where(x >= 0, 1.0 / (1.0 + e), e / (1.0 + e))


def _pass_a_body(x_ref, t_ref, ut_ref, key_ref, c_ref, cpre_ref, hist_ref,
                 cnt_ref, carry_ref):
    b = pl.program_id(0)

    @pl.when(b == 0)
    def _init():
        hist_ref[...] = jnp.zeros_like(hist_ref)
        carry_ref[0] = 0.0
        carry_ref[1] = 0.0

    x = x_ref[0]
    t = t_ref[0].astype(jnp.float32)
    p = _stable_sigmoid(x)
    loss = p - p * t + jnp.log1p(jnp.exp(-p))
    negf = 1.0 - t
    key = jnp.where(negf > 0.5, loss, -1.0)
    key_ref[0] = key

    ut = ut_ref[...]
    incl = jnp.dot(negf, ut, preferred_element_type=jnp.float32)
    rowtot = incl[:, LANE - 1:LANE]
    r_i = lax.broadcasted_iota(jnp.int32, (SUB, SUB), 0)
    c_i = lax.broadcasted_iota(jnp.int32, (SUB, SUB), 1)
    slt = (c_i < r_i).astype(jnp.float32)
    rowpref = jnp.dot(slt, rowtot, preferred_element_type=jnp.float32)
    excl = incl - negf + rowpref
    c_ref[0] = (excl + carry_ref[0]).astype(jnp.int32)
    cpre_ref[0] = jnp.zeros((1, 1), jnp.float32) + carry_ref[0]
    carry_ref[0] = carry_ref[0] + rowpref[SUB - 1, 0] + rowtot[SUB - 1, 0]
    carry_ref[1] = carry_ref[1] + jnp.sum(t)

    binid = jnp.clip(((key - LO) * INV_WIDTH).astype(jnp.int32), 0, NBINS - 1)
    hi_b = binid // 16
    lo_b = binid - hi_b * 16
    iota_h = lax.broadcasted_iota(jnp.int32, (16, LANE), 0)
    iota_l = lax.broadcasted_iota(jnp.int32, (LANE, 16), 1)
    acc = jnp.zeros((16, 16), jnp.float32)
    for r in range(SUB):
        hr = hi_b[r:r + 1, :]
        lr = lo_b[r:r + 1, :]
        nr = negf[r:r + 1, :]
        oh_hi = jnp.where(hr == iota_h, nr, 0.0)
        oh_lo = (lr.reshape(LANE, 1) == iota_l).astype(jnp.float32)
        acc = acc + jnp.dot(oh_hi, oh_lo, preferred_element_type=jnp.float32)
    hist_ref[...] = hist_ref[...] + acc

    @pl.when(b == pl.num_programs(0) - 1)
    def _fin():
        cnt_ref[...] = jnp.zeros((1, 1), jnp.float32) + carry_ref[1]


def _make_sc_compact(S):
    info = plsc.get_sparse_core_info()
    ns = info.num_subcores
    half = S // 2
    per_w = half // ns
    CHUNK = 2048
    nch = per_w // CHUNK
    assert per_w % CHUNK == 0
    SH = half + 16
    mesh = plsc.VectorSubcoreMesh(core_axis_name="c", subcore_axis_name="s")

    @functools.partial(
        pl.kernel,
        mesh=mesh,
        out_type=jax.ShapeDtypeStruct((2, SH), jnp.float32),
        scratch_types=[
            pltpu.VMEM((CHUNK,), jnp.float32),
            pltpu.VMEM((CHUNK,), jnp.int32),
            pltpu.VMEM((16, 128), jnp.int32),
            pltpu.VMEM((16, 128), jnp.float32),
            pltpu.VMEM((32,), jnp.int32),
            pltpu.VMEM_SHARED((SH,), jnp.float32),
            pltpu.SemaphoreType.DMA,
        ],
    )
    def sc_compact(key_hbm, c_hbm, bases_hbm, out_hbm, kv, cv, idx2, val2,
                   bv, shared, sem):
        cid = lax.axis_index("c")
        sid = lax.axis_index("s")
        base = cid * half + sid * per_w

        pltpu.sync_copy(bases_hbm, bv)
        vb = bv[pl.ds(cid * 16, 16)]

        def chunk(ch, carry):
            off = base + ch * CHUNK
            pltpu.sync_copy(key_hbm.at[pl.ds(off, CHUNK)], kv)
            pltpu.sync_copy(c_hbm.at[pl.ds(off, CHUNK)], cv)
            for i in range(CHUNK // 16):
                k16 = kv[pl.ds(i * 16, 16)]
                c16 = cv[pl.ds(i * 16, 16)]
                mi = jnp.where(k16 >= 0.0, c16 - vb, SH - 8)
                r = (i * 16) // 128
                col = (i * 16) % 128
                idx2[r, pl.ds(col, 16)] = mi
                val2[r, pl.ds(col, 16)] = k16
            cps = [pltpu.async_copy(val2.at[r], shared.at[idx2.at[r]], sem)
                   for r in range(16)]
            for cp in cps:
                cp.wait()
            return carry

        lax.fori_loop(0, nch, chunk, 0, unroll=False)
        plsc.subcore_barrier()

        @pl.when(sid == 0)
        def _writeback():
            pltpu.sync_copy(shared, out_hbm.at[cid])

    return sc_compact


def _pass_d_body(x_ref, t_ref, kc_ref, thr_ref, lim_ref, ut_ref, acc_ref,
                 carry_ref):
    b = pl.program_id(0)

    @pl.when(b == 0)
    def _init():
        acc_ref[...] = jnp.zeros_like(acc_ref)
        carry_ref[0] = 0.0

    x = x_ref[0]
    t = t_ref[0].astype(jnp.float32)
    kc = kc_ref[0]
    p = _stable_sigmoid(x)
    loss = p - p * t + jnp.log1p(jnp.exp(-p))
    ps = 1.0 / (1.0 + jnp.exp(-p))
    posf = t

    n_neg = lim_ref[0]
    n_hns = lim_ref[1]

    r_i = lax.broadcasted_iota(jnp.int32, (SUB, LANE), 0).astype(jnp.float32)
    l_i = lax.broadcasted_iota(jnp.int32, (SUB, LANE), 1).astype(jnp.float32)
    fpos = b.astype(jnp.float32) * BLK + r_i * LANE + l_i

    negslot = fpos < n_neg
    valid = negslot & (kc > thr_ref[11])
    validf = jnp.where(valid, 1.0, 0.0)

    segn = jnp.zeros((SUB, LANE), jnp.float32)
    for k in range(11):
        segn = segn + jnp.where(kc <= thr_ref[k], 1.0, 0.0)

    ut = ut_ref[...]
    incl = jnp.dot(posf, ut, preferred_element_type=jnp.float32)
    rowtot = incl[:, LANE - 1:LANE]
    rr = lax.broadcasted_iota(jnp.int32, (SUB, SUB), 0)
    cc = lax.broadcasted_iota(jnp.int32, (SUB, SUB), 1)
    slt = (cc < rr).astype(jnp.float32)
    rowpref = jnp.dot(slt, rowtot, preferred_element_type=jnp.float32)
    pi = incl - posf + rowpref + carry_ref[0]
    carry_ref[0] = carry_ref[0] + rowpref[SUB - 1, 0] + rowtot[SUB - 1, 0]

    slotp = n_hns + pi
    segp = jnp.zeros((SUB, LANE), jnp.float32)
    for k in range(11):
        segp = segp + jnp.where(slotp >= lim_ref[2 + k], 1.0, 0.0)

    a_v = ps * t
    for s in range(12):
        wn = jnp.where(valid & (segn == s), 1.0, 0.0)
        wp = jnp.where((posf > 0.5) & (segp == s), 1.0, 0.0)
        w = wn + wp
        acc_ref[s:s + 1, :] += jnp.sum(a_v * w, axis=0, keepdims=True)
        acc_ref[12 + s:13 + s, :] += jnp.sum(ps * w, axis=0, keepdims=True)
        acc_ref[24 + s:25 + s, :] += jnp.sum(t * w, axis=0, keepdims=True)
    acc_ref[36:37, :] += jnp.sum(loss * (validf + posf), axis=0, keepdims=True)


def kernel(preds, targs):
    bsz, ch = preds.shape[0], preds.shape[1]
    bc = bsz * ch
    S = preds.size
    nblk = S // BLK
    assert S % BLK == 0

    x3 = preds.reshape(nblk, SUB, LANE)
    t3 = targs.reshape(nblk, SUB, LANE)

    li = lax.broadcasted_iota(jnp.int32, (LANE, LANE), 0)
    lj = lax.broadcasted_iota(jnp.int32, (LANE, LANE), 1)
    ut512 = (li <= lj).astype(jnp.float32)

    key3, c3, cpre, hist, npos_arr = pl.pallas_call(
        _pass_a_body,
        grid=(nblk,),
        in_specs=[
            pl.BlockSpec((1, SUB, LANE), lambda b: (b, 0, 0)),
            pl.BlockSpec((1, SUB, LANE), lambda b: (b, 0, 0)),
            pl.BlockSpec((LANE, LANE), lambda b: (0, 0)),
        ],
        out_specs=[
            pl.BlockSpec((1, SUB, LANE), lambda b: (b, 0, 0)),
            pl.BlockSpec((1, SUB, LANE), lambda b: (b, 0, 0)),
            pl.BlockSpec((1, 1, 1), lambda b: (b, 0, 0)),
            pl.BlockSpec((16, 16), lambda b: (0, 0)),
            pl.BlockSpec((1, 1), lambda b: (0, 0)),
        ],
        out_shape=[
            jax.ShapeDtypeStruct((nblk, SUB, LANE), jnp.float32),
            jax.ShapeDtypeStruct((nblk, SUB, LANE), jnp.int32),
            jax.ShapeDtypeStruct((nblk, 1, 1), jnp.float32),
            jax.ShapeDtypeStruct((16, 16), jnp.float32),
            jax.ShapeDtypeStruct((1, 1), jnp.float32),
        ],
        scratch_shapes=[pltpu.SMEM((2,), jnp.float32)],
    )(x3, t3, ut512)

    n_pos = npos_arr[0, 0].astype(jnp.int32)
    n_neg = jnp.int32(S) - n_pos
    n_hns = jnp.where(
        n_pos == 0,
        jnp.floor(0.1 * n_neg.astype(jnp.float32)).astype(jnp.int32),
        jnp.minimum(n_pos * 3, n_neg),
    )
    n_total = n_hns + n_pos
    n_needed = jnp.mod(n_total, bc)
    n_final = n_total + n_needed
    m = jnp.maximum(n_final // bc, 1)

    hist_flat = hist.reshape(NBINS)
    suffix = jnp.cumsum(hist_flat[::-1])[::-1]
    suffix = jnp.concatenate([suffix, jnp.zeros(1, jnp.float32)])
    edges = LO + WIDTH * jnp.arange(NBINS + 1, dtype=jnp.float32)

    def thresh_for(bv):
        h = jnp.argmax(suffix <= bv.astype(jnp.float32))
        return edges[h]

    ks = []
    for k in range(1, 12):
        bv = k * m
        ks.append(jnp.where(bv >= n_hns, jnp.float32(-2.0), thresh_for(bv)))
    k_cut = jnp.where(n_hns >= n_neg, jnp.float32(-0.5), thresh_for(n_hns))
    thr = jnp.stack(ks + [k_cut] + [jnp.float32(0.0)] * 4)

    lim = jnp.concatenate([
        jnp.stack([n_neg.astype(jnp.float32), n_hns.astype(jnp.float32)]),
        (jnp.arange(1, 12, dtype=jnp.float32) * m.astype(jnp.float32)),
        jnp.zeros(3, jnp.float32),
    ])

    half = S // 2
    h_mid = cpre.reshape(nblk)[nblk // 2].astype(jnp.int32)
    bases = jnp.broadcast_to(jnp.stack([jnp.int32(0), h_mid])[:, None],
                             (2, 16)).astype(jnp.int32).reshape(32)
    kc2 = _make_sc_compact(S)(key3.reshape(S), c3.reshape(S), bases)
    SH = half + 16
    pad = S + 1024 - SH
    kc0 = jnp.concatenate([kc2[0], jnp.zeros(pad, jnp.float32)])
    kc1 = jnp.concatenate([kc2[1], jnp.zeros(pad, jnp.float32)])
    arS = jnp.arange(S + 1024, dtype=jnp.int32)
    kc_big = jnp.where(arS < h_mid, kc0, jnp.roll(kc1, h_mid))
    kc3 = kc_big[:S].reshape(nblk, SUB, LANE)

    acc = pl.pallas_call(
        _pass_d_body,
        grid=(nblk,),
        in_specs=[
            pl.BlockSpec((1, SUB, LANE), lambda b: (b, 0, 0)),
            pl.BlockSpec((1, SUB, LANE), lambda b: (b, 0, 0)),
            pl.BlockSpec((1, SUB, LANE), lambda b: (b, 0, 0)),
            pl.BlockSpec(memory_space=pltpu.SMEM),
            pl.BlockSpec(memory_space=pltpu.SMEM),
            pl.BlockSpec((LANE, LANE), lambda b: (0, 0)),
        ],
        out_specs=pl.BlockSpec((48, LANE), lambda b: (0, 0)),
        out_shape=jax.ShapeDtypeStruct((48, LANE), jnp.float32),
        scratch_shapes=[pltpu.SMEM((1,), jnp.float32)],
    )(x3, t3, kc3, thr, lim, ut512)

    sums = jnp.sum(acc, axis=1)
    inter = sums[0:12]
    sump = sums[12:24]
    sumt = sums[24:36]
    lsum = sums[36]

    dice_loss = 1.0 - (2.0 * inter + EPS) / (sump + sumt + EPS)
    dice = jnp.mean(dice_loss)
    mean_l = lsum / n_final.astype(jnp.float32)
    return dice + mean_l
